# Initial kernel scaffold; baseline (speedup 1.0000x reference)
import jax
import jax.numpy as jnp
from jax.experimental import pallas as pl


def kernel(x, edge_index, W_gnn, b_gnn, W1, b1, W2, b2, ln1_g, ln1_b, ln2_g, ln2_b):
    raise NotImplementedError("write your pallas kernel here")


# speedup vs baseline: 15.5114x; 15.5114x over previous
import functools

import jax
import jax.numpy as jnp
from jax import lax
from jax.experimental import pallas as pl
from jax.experimental.pallas import tpu as pltpu
from jax.experimental.pallas import tpu_sc as plsc

N = 10000
E = 320000
D = 128
FF = 256

NC = 2
NS = 16
CHUNK = 80
NCHUNKS = E // CHUNK
CH_T = NCHUNKS // (NC * NS)
RB = 2000
NB = N // RB

_mesh = plsc.VectorSubcoreMesh(core_axis_name="c", subcore_axis_name="s")


@functools.partial(
    pl.kernel,
    out_type=jax.ShapeDtypeStruct((NC, N), jnp.float32),
    mesh=_mesh,
    scratch_types=[
        pltpu.VMEM((CHUNK,), jnp.int32),
        pltpu.VMEM((CHUNK,), jnp.float32),
        pltpu.VMEM_SHARED((N,), jnp.float32),
    ],
)
def _sc_degree(dst_hbm, ones_hbm, zeros_hbm, out_hbm, idx_v, ones_v, deg_sh):
    c = lax.axis_index("c")
    s = lax.axis_index("s")
    wid = c * NS + s

    pltpu.sync_copy(ones_hbm, ones_v)

    @pl.when(s == 0)
    def _():
        pltpu.sync_copy(zeros_hbm, deg_sh)
    plsc.subcore_barrier()

    @pl.loop(0, CH_T)
    def _(j):
        base = (wid * CH_T + j) * CHUNK
        pltpu.sync_copy(dst_hbm.at[pl.ds(base, CHUNK)], idx_v)
        pltpu.sync_copy(ones_v, deg_sh.at[idx_v], add=True)

    plsc.subcore_barrier()

    @pl.when(s == 0)
    def _():
        pltpu.sync_copy(deg_sh, out_hbm.at[c])


@functools.partial(
    pl.kernel,
    out_type=jax.ShapeDtypeStruct((NC, N, D), jnp.float32),
    mesh=_mesh,
    scratch_types=[
        pltpu.VMEM((CHUNK,), jnp.int32),
        pltpu.VMEM((CHUNK,), jnp.int32),
        pltpu.VMEM((CHUNK, D), jnp.float32),
        pltpu.VMEM_SHARED((N, D), jnp.float32),
        pltpu.SemaphoreType.DMA,
    ],
)
def _sc_scatter(hs_hbm, src_hbm, dst_hbm, zeros_hbm, out_hbm,
                src_v, dst_v, rows_v, acc_sh, sem):
    c = lax.axis_index("c")
    s = lax.axis_index("s")
    wid = c * NS + s

    @pl.when(s == 0)
    def _():
        pltpu.sync_copy(zeros_hbm, acc_sh)
    plsc.subcore_barrier()

    @pl.loop(0, CH_T)
    def _(j):
        base = (wid * CH_T + j) * CHUNK
        pltpu.sync_copy(src_hbm.at[pl.ds(base, CHUNK)], src_v)
        pltpu.sync_copy(dst_hbm.at[pl.ds(base, CHUNK)], dst_v)
        pltpu.async_copy(hs_hbm.at[src_v], rows_v, sem).wait()
        pltpu.sync_copy(rows_v, acc_sh.at[dst_v], add=True)

    plsc.subcore_barrier()

    @pl.when(s == 0)
    def _():
        pltpu.sync_copy(acc_sh, out_hbm.at[c])


def _dot(a, b):
    return lax.dot_general(a, b, (((1,), (0,)), ((), ())),
                           precision=lax.Precision.HIGHEST,
                           preferred_element_type=jnp.float32)


def _mm_body(x_ref, w_ref, o_ref):
    o_ref[...] = _dot(x_ref[...], w_ref[...])


_mm = pl.pallas_call(
    _mm_body,
    grid=(NB,),
    in_specs=[pl.BlockSpec((RB, D), lambda i: (i, 0)),
              pl.BlockSpec((D, D), lambda i: (0, 0))],
    out_specs=pl.BlockSpec((RB, D), lambda i: (i, 0)),
    out_shape=jax.ShapeDtypeStruct((N, D), jnp.float32),
)


def _scale_body(h_ref, deg_ref, o_ref):
    dinv = lax.rsqrt(jnp.maximum(deg_ref[...] + 1.0, 1.0))
    o_ref[...] = h_ref[...] * dinv


_scale = pl.pallas_call(
    _scale_body,
    grid=(NB,),
    in_specs=[pl.BlockSpec((RB, D), lambda i: (i, 0)),
              pl.BlockSpec((RB, 1), lambda i: (i, 0))],
    out_specs=pl.BlockSpec((RB, D), lambda i: (i, 0)),
    out_shape=jax.ShapeDtypeStruct((N, D), jnp.float32),
)


def _final_body(x_ref, hs_ref, accp_ref, deg_ref, bg_ref, w1_ref, b1_ref,
                w2_ref, b2_ref, g1_ref, bb1_ref, g2_ref, bb2_ref, o_ref):
    dinv = lax.rsqrt(jnp.maximum(deg_ref[...] + 1.0, 1.0))
    g = (accp_ref[0] + accp_ref[1] + hs_ref[...]) * dinv + bg_ref[...]
    x1 = x_ref[...] + g
    mu = jnp.mean(x1, axis=1, keepdims=True)
    var = jnp.mean(jnp.square(x1 - mu), axis=1, keepdims=True)
    xn = (x1 - mu) * lax.rsqrt(var + 1e-5) * g1_ref[...] + bb1_ref[...]
    t = jnp.maximum(_dot(xn, w1_ref[...]) + b1_ref[...], 0.0)
    x2 = xn + _dot(t, w2_ref[...]) + b2_ref[...]
    mu2 = jnp.mean(x2, axis=1, keepdims=True)
    var2 = jnp.mean(jnp.square(x2 - mu2), axis=1, keepdims=True)
    o_ref[...] = (x2 - mu2) * lax.rsqrt(var2 + 1e-5) * g2_ref[...] + bb2_ref[...]


_final = pl.pallas_call(
    _final_body,
    grid=(NB,),
    in_specs=[pl.BlockSpec((RB, D), lambda i: (i, 0)),
              pl.BlockSpec((RB, D), lambda i: (i, 0)),
              pl.BlockSpec((NC, RB, D), lambda i: (0, i, 0)),
              pl.BlockSpec((RB, 1), lambda i: (i, 0)),
              pl.BlockSpec((1, D), lambda i: (0, 0)),
              pl.BlockSpec((D, FF), lambda i: (0, 0)),
              pl.BlockSpec((1, FF), lambda i: (0, 0)),
              pl.BlockSpec((FF, D), lambda i: (0, 0)),
              pl.BlockSpec((1, D), lambda i: (0, 0)),
              pl.BlockSpec((1, D), lambda i: (0, 0)),
              pl.BlockSpec((1, D), lambda i: (0, 0)),
              pl.BlockSpec((1, D), lambda i: (0, 0)),
              pl.BlockSpec((1, D), lambda i: (0, 0))],
    out_specs=pl.BlockSpec((RB, D), lambda i: (i, 0)),
    out_shape=jax.ShapeDtypeStruct((N, D), jnp.float32),
)


def kernel(x, edge_index, W_gnn, b_gnn, W1, b1, W2, b2,
           ln1_g, ln1_b, ln2_g, ln2_b):
    src = edge_index[0]
    dst = edge_index[1]
    zeros_nd = jnp.zeros((N, D), jnp.float32)
    degp = _sc_degree(dst, jnp.ones((CHUNK,), jnp.float32),
                      jnp.zeros((N,), jnp.float32))
    deg2d = (degp[0] + degp[1]).reshape(N, 1)
    h = _mm(x, W_gnn)
    hs = _scale(h, deg2d)
    accp = _sc_scatter(hs, src, dst, zeros_nd)
    return _final(x, hs, accp, deg2d, b_gnn.reshape(1, D), W1,
                  b1.reshape(1, FF), W2, b2.reshape(1, D),
                  ln1_g.reshape(1, D), ln1_b.reshape(1, D),
                  ln2_g.reshape(1, D), ln2_b.reshape(1, D))


# speedup vs baseline: 19.7487x; 1.2732x over previous
import functools

import jax
import jax.numpy as jnp
from jax import lax
from jax.experimental import pallas as pl
from jax.experimental.pallas import tpu as pltpu
from jax.experimental.pallas import tpu_sc as plsc

N = 10000
E = 320000
D = 128
FF = 256

NC = 2
NS = 16
CHUNK = 80
NCHUNKS = E // CHUNK
CH_T = NCHUNKS // (NC * NS)
RB = 2000
NB = N // RB

_mesh = plsc.VectorSubcoreMesh(core_axis_name="c", subcore_axis_name="s")


@functools.partial(
    pl.kernel,
    out_type=jax.ShapeDtypeStruct((NC, N), jnp.float32),
    mesh=_mesh,
    scratch_types=[
        pltpu.VMEM((CHUNK,), jnp.int32),
        pltpu.VMEM((CHUNK,), jnp.float32),
        pltpu.VMEM_SHARED((N,), jnp.float32),
    ],
)
def _sc_degree(dst_hbm, ones_hbm, zeros_hbm, out_hbm, idx_v, ones_v, deg_sh):
    c = lax.axis_index("c")
    s = lax.axis_index("s")
    wid = c * NS + s

    pltpu.sync_copy(ones_hbm, ones_v)

    @pl.when(s == 0)
    def _():
        pltpu.sync_copy(zeros_hbm, deg_sh)
    plsc.subcore_barrier()

    @pl.loop(0, CH_T)
    def _(j):
        base = (wid * CH_T + j) * CHUNK
        pltpu.sync_copy(dst_hbm.at[pl.ds(base, CHUNK)], idx_v)
        pltpu.sync_copy(ones_v, deg_sh.at[idx_v], add=True)

    plsc.subcore_barrier()

    @pl.when(s == 0)
    def _():
        pltpu.sync_copy(deg_sh, out_hbm.at[c])


@functools.partial(
    pl.kernel,
    out_type=jax.ShapeDtypeStruct((NC, N, D), jnp.float32),
    mesh=_mesh,
    scratch_types=[
        pltpu.VMEM((CHUNK,), jnp.int32),
        pltpu.VMEM((CHUNK,), jnp.int32),
        pltpu.VMEM((CHUNK, D), jnp.float32),
        pltpu.VMEM((CHUNK,), jnp.int32),
        pltpu.VMEM((CHUNK,), jnp.int32),
        pltpu.VMEM((CHUNK, D), jnp.float32),
        pltpu.VMEM_SHARED((N, D), jnp.float32),
        pltpu.SemaphoreType.DMA,
        pltpu.SemaphoreType.DMA,
        pltpu.SemaphoreType.DMA,
        pltpu.SemaphoreType.DMA,
        pltpu.SemaphoreType.DMA,
        pltpu.SemaphoreType.DMA,
    ],
)
def _sc_scatter(hs_hbm, src_hbm, dst_hbm, zeros_hbm, out_hbm,
                src0, dst0, rows0, src1, dst1, rows1, acc_sh,
                isem0, gsem0, ssem0, isem1, gsem1, ssem1):
    c = lax.axis_index("c")
    s = lax.axis_index("s")
    wid = c * NS + s
    base0 = wid * CH_T * CHUNK

    @pl.when(s == 0)
    def _():
        pltpu.sync_copy(zeros_hbm, acc_sh)
    plsc.subcore_barrier()

    bufs = ((src0, dst0, rows0, isem0, gsem0, ssem0),
            (src1, dst1, rows1, isem1, gsem1, ssem1))

    @pl.loop(0, (CH_T + 1) // 2)
    def _(i):
        for b in (0, 1):
            src_v, dst_v, rows_v, isem, gsem, ssem = bufs[b]
            j = 2 * i + b

            @pl.when(j < CH_T)
            def _():
                base = base0 + j * CHUNK

                @pl.when(j >= 2)
                def _():
                    pltpu.make_async_copy(rows_v, acc_sh.at[dst_v], ssem).wait()

                pltpu.async_copy(src_hbm.at[pl.ds(base, CHUNK)], src_v, isem)
                pltpu.async_copy(dst_hbm.at[pl.ds(base, CHUNK)], dst_v, isem)
                pltpu.make_async_copy(src_hbm.at[pl.ds(base, CHUNK)],
                                      src_v, isem).wait()
                pltpu.make_async_copy(dst_hbm.at[pl.ds(base, CHUNK)],
                                      dst_v, isem).wait()
                pltpu.async_copy(hs_hbm.at[src_v], rows_v, gsem).wait()
                pltpu.async_copy(rows_v, acc_sh.at[dst_v], ssem, add=True)

    pltpu.make_async_copy(rows0, acc_sh.at[dst0], ssem0).wait()
    pltpu.make_async_copy(rows1, acc_sh.at[dst1], ssem1).wait()

    plsc.subcore_barrier()

    @pl.when(s == 0)
    def _():
        pltpu.sync_copy(acc_sh, out_hbm.at[c])


def _dot(a, b):
    return lax.dot_general(a, b, (((1,), (0,)), ((), ())),
                           precision=lax.Precision.HIGHEST,
                           preferred_element_type=jnp.float32)


def _mm_body(x_ref, w_ref, o_ref):
    o_ref[...] = _dot(x_ref[...], w_ref[...])


_mm = pl.pallas_call(
    _mm_body,
    grid=(NB,),
    in_specs=[pl.BlockSpec((RB, D), lambda i: (i, 0)),
              pl.BlockSpec((D, D), lambda i: (0, 0))],
    out_specs=pl.BlockSpec((RB, D), lambda i: (i, 0)),
    out_shape=jax.ShapeDtypeStruct((N, D), jnp.float32),
)


def _scale_body(h_ref, deg_ref, o_ref):
    dinv = lax.rsqrt(jnp.maximum(deg_ref[...] + 1.0, 1.0))
    o_ref[...] = h_ref[...] * dinv


_scale = pl.pallas_call(
    _scale_body,
    grid=(NB,),
    in_specs=[pl.BlockSpec((RB, D), lambda i: (i, 0)),
              pl.BlockSpec((RB, 1), lambda i: (i, 0))],
    out_specs=pl.BlockSpec((RB, D), lambda i: (i, 0)),
    out_shape=jax.ShapeDtypeStruct((N, D), jnp.float32),
)


def _final_body(x_ref, hs_ref, accp_ref, deg_ref, bg_ref, w1_ref, b1_ref,
                w2_ref, b2_ref, g1_ref, bb1_ref, g2_ref, bb2_ref, o_ref):
    dinv = lax.rsqrt(jnp.maximum(deg_ref[...] + 1.0, 1.0))
    g = (accp_ref[0] + accp_ref[1] + hs_ref[...]) * dinv + bg_ref[...]
    x1 = x_ref[...] + g
    mu = jnp.mean(x1, axis=1, keepdims=True)
    var = jnp.mean(jnp.square(x1 - mu), axis=1, keepdims=True)
    xn = (x1 - mu) * lax.rsqrt(var + 1e-5) * g1_ref[...] + bb1_ref[...]
    t = jnp.maximum(_dot(xn, w1_ref[...]) + b1_ref[...], 0.0)
    x2 = xn + _dot(t, w2_ref[...]) + b2_ref[...]
    mu2 = jnp.mean(x2, axis=1, keepdims=True)
    var2 = jnp.mean(jnp.square(x2 - mu2), axis=1, keepdims=True)
    o_ref[...] = (x2 - mu2) * lax.rsqrt(var2 + 1e-5) * g2_ref[...] + bb2_ref[...]


_final = pl.pallas_call(
    _final_body,
    grid=(NB,),
    in_specs=[pl.BlockSpec((RB, D), lambda i: (i, 0)),
              pl.BlockSpec((RB, D), lambda i: (i, 0)),
              pl.BlockSpec((NC, RB, D), lambda i: (0, i, 0)),
              pl.BlockSpec((RB, 1), lambda i: (i, 0)),
              pl.BlockSpec((1, D), lambda i: (0, 0)),
              pl.BlockSpec((D, FF), lambda i: (0, 0)),
              pl.BlockSpec((1, FF), lambda i: (0, 0)),
              pl.BlockSpec((FF, D), lambda i: (0, 0)),
              pl.BlockSpec((1, D), lambda i: (0, 0)),
              pl.BlockSpec((1, D), lambda i: (0, 0)),
              pl.BlockSpec((1, D), lambda i: (0, 0)),
              pl.BlockSpec((1, D), lambda i: (0, 0)),
              pl.BlockSpec((1, D), lambda i: (0, 0))],
    out_specs=pl.BlockSpec((RB, D), lambda i: (i, 0)),
    out_shape=jax.ShapeDtypeStruct((N, D), jnp.float32),
)


def kernel(x, edge_index, W_gnn, b_gnn, W1, b1, W2, b2,
           ln1_g, ln1_b, ln2_g, ln2_b):
    src = edge_index[0]
    dst = edge_index[1]
    zeros_nd = jnp.zeros((N, D), jnp.float32)
    degp = _sc_degree(dst, jnp.ones((CHUNK,), jnp.float32),
                      jnp.zeros((N,), jnp.float32))
    deg2d = (degp[0] + degp[1]).reshape(N, 1)
    h = _mm(x, W_gnn)
    hs = _scale(h, deg2d)
    accp = _sc_scatter(hs, src, dst, zeros_nd)
    return _final(x, hs, accp, deg2d, b_gnn.reshape(1, D), W1,
                  b1.reshape(1, FF), W2, b2.reshape(1, D),
                  ln1_g.reshape(1, D), ln1_b.reshape(1, D),
                  ln2_g.reshape(1, D), ln2_b.reshape(1, D))


# speedup vs baseline: 20.3284x; 1.0294x over previous
import functools

import jax
import jax.numpy as jnp
from jax import lax
from jax.experimental import pallas as pl
from jax.experimental.pallas import tpu as pltpu
from jax.experimental.pallas import tpu_sc as plsc

N = 10000
E = 320000
D = 128
FF = 256

NC = 2
NS = 16
CHUNK = 80
NCHUNKS = E // CHUNK
CH_T = NCHUNKS // (NC * NS)
RB = 2000
NB = N // RB

_mesh = plsc.VectorSubcoreMesh(core_axis_name="c", subcore_axis_name="s")


@functools.partial(
    pl.kernel,
    out_type=jax.ShapeDtypeStruct((NC, N), jnp.float32),
    mesh=_mesh,
    scratch_types=[
        pltpu.VMEM((CHUNK,), jnp.int32),
        pltpu.VMEM((CHUNK,), jnp.int32),
        pltpu.VMEM((CHUNK,), jnp.float32),
        pltpu.VMEM_SHARED((N,), jnp.float32),
        pltpu.SemaphoreType.DMA,
        pltpu.SemaphoreType.DMA,
        pltpu.SemaphoreType.DMA,
        pltpu.SemaphoreType.DMA,
    ],
)
def _sc_degree(dst_hbm, ones_hbm, zeros_hbm, out_hbm,
               idx0, idx1, ones_v, deg_sh, isem0, ssem0, isem1, ssem1):
    c = lax.axis_index("c")
    s = lax.axis_index("s")
    wid = c * NS + s
    base0 = wid * CH_T * CHUNK

    pltpu.sync_copy(ones_hbm, ones_v)

    @pl.when(s == 0)
    def _():
        pltpu.sync_copy(zeros_hbm, deg_sh)
    plsc.subcore_barrier()

    bufs = ((idx0, isem0, ssem0), (idx1, isem1, ssem1))

    @pl.loop(0, (CH_T + 1) // 2)
    def _(i):
        for b in (0, 1):
            idx_v, isem, ssem = bufs[b]
            j = 2 * i + b

            @pl.when(j < CH_T)
            def _():
                base = base0 + j * CHUNK

                @pl.when(j >= 2)
                def _():
                    pltpu.make_async_copy(ones_v, deg_sh.at[idx_v], ssem).wait()

                pltpu.async_copy(dst_hbm.at[pl.ds(base, CHUNK)],
                                 idx_v, isem).wait()
                pltpu.async_copy(ones_v, deg_sh.at[idx_v], ssem, add=True)

    pltpu.make_async_copy(ones_v, deg_sh.at[idx0], ssem0).wait()
    pltpu.make_async_copy(ones_v, deg_sh.at[idx1], ssem1).wait()

    plsc.subcore_barrier()

    @pl.when(s == 0)
    def _():
        pltpu.sync_copy(deg_sh, out_hbm.at[c])


@functools.partial(
    pl.kernel,
    out_type=jax.ShapeDtypeStruct((NC, N, D), jnp.float32),
    mesh=_mesh,
    scratch_types=[
        pltpu.VMEM((CHUNK,), jnp.int32),
        pltpu.VMEM((CHUNK,), jnp.int32),
        pltpu.VMEM((CHUNK, D), jnp.float32),
        pltpu.VMEM((CHUNK,), jnp.int32),
        pltpu.VMEM((CHUNK,), jnp.int32),
        pltpu.VMEM((CHUNK, D), jnp.float32),
        pltpu.VMEM_SHARED((N, D), jnp.float32),
        pltpu.SemaphoreType.DMA,
        pltpu.SemaphoreType.DMA,
        pltpu.SemaphoreType.DMA,
        pltpu.SemaphoreType.DMA,
        pltpu.SemaphoreType.DMA,
        pltpu.SemaphoreType.DMA,
    ],
)
def _sc_scatter(hs_hbm, src_hbm, dst_hbm, zeros_hbm, out_hbm,
                src0, dst0, rows0, src1, dst1, rows1, acc_sh,
                isem0, gsem0, ssem0, isem1, gsem1, ssem1):
    c = lax.axis_index("c")
    s = lax.axis_index("s")
    wid = c * NS + s
    base0 = wid * CH_T * CHUNK

    @pl.when(s == 0)
    def _():
        pltpu.sync_copy(zeros_hbm, acc_sh)
    plsc.subcore_barrier()

    bufs = ((src0, dst0, rows0, isem0, gsem0, ssem0),
            (src1, dst1, rows1, isem1, gsem1, ssem1))

    @pl.loop(0, (CH_T + 1) // 2)
    def _(i):
        for b in (0, 1):
            src_v, dst_v, rows_v, isem, gsem, ssem = bufs[b]
            j = 2 * i + b

            @pl.when(j < CH_T)
            def _():
                base = base0 + j * CHUNK

                @pl.when(j >= 2)
                def _():
                    pltpu.make_async_copy(rows_v, acc_sh.at[dst_v], ssem).wait()

                pltpu.async_copy(src_hbm.at[pl.ds(base, CHUNK)], src_v, isem)
                pltpu.async_copy(dst_hbm.at[pl.ds(base, CHUNK)], dst_v, isem)
                pltpu.make_async_copy(src_hbm.at[pl.ds(base, CHUNK)],
                                      src_v, isem).wait()
                pltpu.make_async_copy(dst_hbm.at[pl.ds(base, CHUNK)],
                                      dst_v, isem).wait()
                pltpu.async_copy(hs_hbm.at[src_v], rows_v, gsem).wait()
                pltpu.async_copy(rows_v, acc_sh.at[dst_v], ssem, add=True)

    pltpu.make_async_copy(rows0, acc_sh.at[dst0], ssem0).wait()
    pltpu.make_async_copy(rows1, acc_sh.at[dst1], ssem1).wait()

    plsc.subcore_barrier()

    @pl.when(s == 0)
    def _():
        pltpu.sync_copy(acc_sh, out_hbm.at[c])


def _dot(a, b):
    return lax.dot_general(a, b, (((1,), (0,)), ((), ())),
                           precision=lax.Precision.HIGHEST,
                           preferred_element_type=jnp.float32)


def _mm_body(x_ref, w_ref, o_ref):
    o_ref[...] = _dot(x_ref[...], w_ref[...])


_mm = pl.pallas_call(
    _mm_body,
    grid=(NB,),
    in_specs=[pl.BlockSpec((RB, D), lambda i: (i, 0)),
              pl.BlockSpec((D, D), lambda i: (0, 0))],
    out_specs=pl.BlockSpec((RB, D), lambda i: (i, 0)),
    out_shape=jax.ShapeDtypeStruct((N, D), jnp.float32),
)


def _scale_body(h_ref, deg_ref, o_ref):
    dinv = lax.rsqrt(jnp.maximum(deg_ref[...] + 1.0, 1.0))
    o_ref[...] = h_ref[...] * dinv


_scale = pl.pallas_call(
    _scale_body,
    grid=(NB,),
    in_specs=[pl.BlockSpec((RB, D), lambda i: (i, 0)),
              pl.BlockSpec((RB, 1), lambda i: (i, 0))],
    out_specs=pl.BlockSpec((RB, D), lambda i: (i, 0)),
    out_shape=jax.ShapeDtypeStruct((N, D), jnp.float32),
)


def _final_body(x_ref, hs_ref, accp_ref, deg_ref, bg_ref, w1_ref, b1_ref,
                w2_ref, b2_ref, g1_ref, bb1_ref, g2_ref, bb2_ref, o_ref):
    dinv = lax.rsqrt(jnp.maximum(deg_ref[...] + 1.0, 1.0))
    g = (accp_ref[0] + accp_ref[1] + hs_ref[...]) * dinv + bg_ref[...]
    x1 = x_ref[...] + g
    mu = jnp.mean(x1, axis=1, keepdims=True)
    var = jnp.mean(jnp.square(x1 - mu), axis=1, keepdims=True)
    xn = (x1 - mu) * lax.rsqrt(var + 1e-5) * g1_ref[...] + bb1_ref[...]
    t = jnp.maximum(_dot(xn, w1_ref[...]) + b1_ref[...], 0.0)
    x2 = xn + _dot(t, w2_ref[...]) + b2_ref[...]
    mu2 = jnp.mean(x2, axis=1, keepdims=True)
    var2 = jnp.mean(jnp.square(x2 - mu2), axis=1, keepdims=True)
    o_ref[...] = (x2 - mu2) * lax.rsqrt(var2 + 1e-5) * g2_ref[...] + bb2_ref[...]


_final = pl.pallas_call(
    _final_body,
    grid=(NB,),
    in_specs=[pl.BlockSpec((RB, D), lambda i: (i, 0)),
              pl.BlockSpec((RB, D), lambda i: (i, 0)),
              pl.BlockSpec((NC, RB, D), lambda i: (0, i, 0)),
              pl.BlockSpec((RB, 1), lambda i: (i, 0)),
              pl.BlockSpec((1, D), lambda i: (0, 0)),
              pl.BlockSpec((D, FF), lambda i: (0, 0)),
              pl.BlockSpec((1, FF), lambda i: (0, 0)),
              pl.BlockSpec((FF, D), lambda i: (0, 0)),
              pl.BlockSpec((1, D), lambda i: (0, 0)),
              pl.BlockSpec((1, D), lambda i: (0, 0)),
              pl.BlockSpec((1, D), lambda i: (0, 0)),
              pl.BlockSpec((1, D), lambda i: (0, 0)),
              pl.BlockSpec((1, D), lambda i: (0, 0))],
    out_specs=pl.BlockSpec((RB, D), lambda i: (i, 0)),
    out_shape=jax.ShapeDtypeStruct((N, D), jnp.float32),
)


def kernel(x, edge_index, W_gnn, b_gnn, W1, b1, W2, b2,
           ln1_g, ln1_b, ln2_g, ln2_b):
    src = edge_index[0]
    dst = edge_index[1]
    zeros_nd = jnp.zeros((N, D), jnp.float32)
    degp = _sc_degree(dst, jnp.ones((CHUNK,), jnp.float32),
                      jnp.zeros((N,), jnp.float32))
    deg2d = (degp[0] + degp[1]).reshape(N, 1)
    h = _mm(x, W_gnn)
    hs = _scale(h, deg2d)
    accp = _sc_scatter(hs, src, dst, zeros_nd)
    return _final(x, hs, accp, deg2d, b_gnn.reshape(1, D), W1,
                  b1.reshape(1, FF), W2, b2.reshape(1, D),
                  ln1_g.reshape(1, D), ln1_b.reshape(1, D),
                  ln2_g.reshape(1, D), ln2_b.reshape(1, D))


# speedup vs baseline: 23.2034x; 1.1414x over previous
import functools

import jax
import jax.numpy as jnp
from jax import lax
from jax.experimental import pallas as pl
from jax.experimental.pallas import tpu as pltpu
from jax.experimental.pallas import tpu_sc as plsc

N = 10000
E = 320000
D = 128
FF = 256

NC = 2
NS = 16
CHUNK = 80
NCHUNKS = E // CHUNK
CH_T = NCHUNKS // (NC * NS)
MCH = 128
NCHM = E // MCH
KT = (NCHM + NC * NS - 1) // (NC * NS)
RB = 2000
NB = N // RB

_mesh = plsc.VectorSubcoreMesh(core_axis_name="c", subcore_axis_name="s")


@functools.partial(
    pl.kernel,
    out_type=jax.ShapeDtypeStruct((NC, N), jnp.float32),
    mesh=_mesh,
    scratch_types=[
        pltpu.VMEM((CHUNK,), jnp.int32),
        pltpu.VMEM((CHUNK,), jnp.int32),
        pltpu.VMEM((CHUNK,), jnp.float32),
        pltpu.VMEM_SHARED((N,), jnp.float32),
        pltpu.SemaphoreType.DMA,
        pltpu.SemaphoreType.DMA,
        pltpu.SemaphoreType.DMA,
        pltpu.SemaphoreType.DMA,
    ],
)
def _sc_degree(dst_hbm, ones_hbm, zeros_hbm, out_hbm,
               idx0, idx1, ones_v, deg_sh, isem0, ssem0, isem1, ssem1):
    c = lax.axis_index("c")
    s = lax.axis_index("s")
    wid = c * NS + s
    base0 = wid * CH_T * CHUNK

    pltpu.sync_copy(ones_hbm, ones_v)

    @pl.when(s == 0)
    def _():
        pltpu.sync_copy(zeros_hbm, deg_sh)
    plsc.subcore_barrier()

    bufs = ((idx0, isem0, ssem0), (idx1, isem1, ssem1))

    @pl.loop(0, (CH_T + 1) // 2)
    def _(i):
        for b in (0, 1):
            idx_v, isem, ssem = bufs[b]
            j = 2 * i + b

            @pl.when(j < CH_T)
            def _():
                base = base0 + j * CHUNK

                @pl.when(j >= 2)
                def _():
                    pltpu.make_async_copy(ones_v, deg_sh.at[idx_v], ssem).wait()

                pltpu.async_copy(dst_hbm.at[pl.ds(base, CHUNK)],
                                 idx_v, isem).wait()
                pltpu.async_copy(ones_v, deg_sh.at[idx_v], ssem, add=True)

    pltpu.make_async_copy(ones_v, deg_sh.at[idx0], ssem0).wait()
    pltpu.make_async_copy(ones_v, deg_sh.at[idx1], ssem1).wait()

    plsc.subcore_barrier()

    @pl.when(s == 0)
    def _():
        pltpu.sync_copy(deg_sh, out_hbm.at[c])


@functools.partial(
    pl.kernel,
    out_type=jax.ShapeDtypeStruct((NC, N, D), jnp.float32),
    mesh=_mesh,
    scratch_types=[
        pltpu.VMEM((MCH,), jnp.int32),
        pltpu.VMEM((MCH,), jnp.int32),
        pltpu.VMEM((MCH, D), jnp.float32),
        pltpu.VMEM((MCH,), jnp.int32),
        pltpu.VMEM((MCH,), jnp.int32),
        pltpu.VMEM((MCH, D), jnp.float32),
        pltpu.VMEM_SHARED((N, D), jnp.float32),
        pltpu.SemaphoreType.DMA,
        pltpu.SemaphoreType.DMA,
        pltpu.SemaphoreType.DMA,
        pltpu.SemaphoreType.DMA,
        pltpu.SemaphoreType.DMA,
        pltpu.SemaphoreType.DMA,
    ],
)
def _sc_scatter(hs_hbm, src_hbm, dst_hbm, zeros_hbm, out_hbm,
                src0, dst0, rows0, src1, dst1, rows1, acc_sh,
                isem0, gsem0, ssem0, isem1, gsem1, ssem1):
    c = lax.axis_index("c")
    s = lax.axis_index("s")
    wid = c * NS + s

    @pl.when(s == 0)
    def _():
        pltpu.sync_copy(zeros_hbm, acc_sh)
    plsc.subcore_barrier()

    bufs = ((src0, dst0, rows0, isem0, gsem0, ssem0),
            (src1, dst1, rows1, isem1, gsem1, ssem1))

    @pl.loop(0, (KT + 1) // 2)
    def _(i):
        for b in (0, 1):
            src_v, dst_v, rows_v, isem, gsem, ssem = bufs[b]
            k = 2 * i + b
            ch = wid + (NC * NS) * k

            @pl.when(ch < NCHM)
            def _():
                base = ch * MCH

                @pl.when(k >= 2)
                def _():
                    pltpu.make_async_copy(rows_v, acc_sh.at[dst_v], ssem).wait()

                pltpu.async_copy(src_hbm.at[pl.ds(base, MCH)], src_v, isem)
                pltpu.async_copy(dst_hbm.at[pl.ds(base, MCH)], dst_v, isem)
                pltpu.make_async_copy(src_hbm.at[pl.ds(base, MCH)],
                                      src_v, isem).wait()
                pltpu.make_async_copy(dst_hbm.at[pl.ds(base, MCH)],
                                      dst_v, isem).wait()
                pltpu.async_copy(hs_hbm.at[src_v], rows_v, gsem).wait()
                pltpu.async_copy(rows_v, acc_sh.at[dst_v], ssem, add=True)

    pltpu.make_async_copy(rows0, acc_sh.at[dst0], ssem0).wait()
    pltpu.make_async_copy(rows1, acc_sh.at[dst1], ssem1).wait()

    plsc.subcore_barrier()

    @pl.when(s == 0)
    def _():
        pltpu.sync_copy(acc_sh, out_hbm.at[c])


def _dot(a, b):
    return lax.dot_general(a, b, (((1,), (0,)), ((), ())),
                           precision=lax.Precision.HIGHEST,
                           preferred_element_type=jnp.float32)


def _mm_body(x_ref, w_ref, o_ref):
    o_ref[...] = _dot(x_ref[...], w_ref[...])


_mm = pl.pallas_call(
    _mm_body,
    grid=(NB,),
    in_specs=[pl.BlockSpec((RB, D), lambda i: (i, 0)),
              pl.BlockSpec((D, D), lambda i: (0, 0))],
    out_specs=pl.BlockSpec((RB, D), lambda i: (i, 0)),
    out_shape=jax.ShapeDtypeStruct((N, D), jnp.float32),
)


def _scale_body(h_ref, deg_ref, o_ref):
    dinv = lax.rsqrt(jnp.maximum(deg_ref[...] + 1.0, 1.0))
    o_ref[...] = h_ref[...] * dinv


_scale = pl.pallas_call(
    _scale_body,
    grid=(NB,),
    in_specs=[pl.BlockSpec((RB, D), lambda i: (i, 0)),
              pl.BlockSpec((RB, 1), lambda i: (i, 0))],
    out_specs=pl.BlockSpec((RB, D), lambda i: (i, 0)),
    out_shape=jax.ShapeDtypeStruct((N, D), jnp.float32),
)


def _final_body(x_ref, hs_ref, accp_ref, deg_ref, bg_ref, w1_ref, b1_ref,
                w2_ref, b2_ref, g1_ref, bb1_ref, g2_ref, bb2_ref, o_ref):
    dinv = lax.rsqrt(jnp.maximum(deg_ref[...] + 1.0, 1.0))
    g = (accp_ref[0] + accp_ref[1] + hs_ref[...]) * dinv + bg_ref[...]
    x1 = x_ref[...] + g
    mu = jnp.mean(x1, axis=1, keepdims=True)
    var = jnp.mean(jnp.square(x1 - mu), axis=1, keepdims=True)
    xn = (x1 - mu) * lax.rsqrt(var + 1e-5) * g1_ref[...] + bb1_ref[...]
    t = jnp.maximum(_dot(xn, w1_ref[...]) + b1_ref[...], 0.0)
    x2 = xn + _dot(t, w2_ref[...]) + b2_ref[...]
    mu2 = jnp.mean(x2, axis=1, keepdims=True)
    var2 = jnp.mean(jnp.square(x2 - mu2), axis=1, keepdims=True)
    o_ref[...] = (x2 - mu2) * lax.rsqrt(var2 + 1e-5) * g2_ref[...] + bb2_ref[...]


_final = pl.pallas_call(
    _final_body,
    grid=(NB,),
    in_specs=[pl.BlockSpec((RB, D), lambda i: (i, 0)),
              pl.BlockSpec((RB, D), lambda i: (i, 0)),
              pl.BlockSpec((NC, RB, D), lambda i: (0, i, 0)),
              pl.BlockSpec((RB, 1), lambda i: (i, 0)),
              pl.BlockSpec((1, D), lambda i: (0, 0)),
              pl.BlockSpec((D, FF), lambda i: (0, 0)),
              pl.BlockSpec((1, FF), lambda i: (0, 0)),
              pl.BlockSpec((FF, D), lambda i: (0, 0)),
              pl.BlockSpec((1, D), lambda i: (0, 0)),
              pl.BlockSpec((1, D), lambda i: (0, 0)),
              pl.BlockSpec((1, D), lambda i: (0, 0)),
              pl.BlockSpec((1, D), lambda i: (0, 0)),
              pl.BlockSpec((1, D), lambda i: (0, 0))],
    out_specs=pl.BlockSpec((RB, D), lambda i: (i, 0)),
    out_shape=jax.ShapeDtypeStruct((N, D), jnp.float32),
)


def kernel(x, edge_index, W_gnn, b_gnn, W1, b1, W2, b2,
           ln1_g, ln1_b, ln2_g, ln2_b):
    src = edge_index[0]
    dst = edge_index[1]
    zeros_nd = jnp.zeros((N, D), jnp.float32)
    degp = _sc_degree(dst, jnp.ones((CHUNK,), jnp.float32),
                      jnp.zeros((N,), jnp.float32))
    deg2d = (degp[0] + degp[1]).reshape(N, 1)
    h = _mm(x, W_gnn)
    hs = _scale(h, deg2d)
    accp = _sc_scatter(hs, src, dst, zeros_nd)
    return _final(x, hs, accp, deg2d, b_gnn.reshape(1, D), W1,
                  b1.reshape(1, FF), W2, b2.reshape(1, D),
                  ln1_g.reshape(1, D), ln1_b.reshape(1, D),
                  ln2_g.reshape(1, D), ln2_b.reshape(1, D))


# speedup vs baseline: 25.9317x; 1.1176x over previous
import functools

import jax
import jax.numpy as jnp
from jax import lax
from jax.experimental import pallas as pl
from jax.experimental.pallas import tpu as pltpu
from jax.experimental.pallas import tpu_sc as plsc

N = 10000
E = 320000
D = 128
FF = 256

NC = 2
NS = 16
CHUNK = 80
NCHUNKS = E // CHUNK
CH_T = NCHUNKS // (NC * NS)
MCH = 128
NCHM = E // MCH
KT = (NCHM + NC * NS - 1) // (NC * NS)
RB = 2000
NB = N // RB

_mesh = plsc.VectorSubcoreMesh(core_axis_name="c", subcore_axis_name="s")


@functools.partial(
    pl.kernel,
    out_type=jax.ShapeDtypeStruct((NC, N), jnp.float32),
    mesh=_mesh,
    scratch_types=[
        pltpu.VMEM((CHUNK,), jnp.int32),
        pltpu.VMEM((CHUNK,), jnp.int32),
        pltpu.VMEM((CHUNK,), jnp.float32),
        pltpu.VMEM_SHARED((N,), jnp.float32),
        pltpu.SemaphoreType.DMA,
        pltpu.SemaphoreType.DMA,
        pltpu.SemaphoreType.DMA,
        pltpu.SemaphoreType.DMA,
    ],
)
def _sc_degree(dst_hbm, ones_hbm, zeros_hbm, out_hbm,
               idx0, idx1, ones_v, deg_sh, isem0, ssem0, isem1, ssem1):
    c = lax.axis_index("c")
    s = lax.axis_index("s")
    wid = c * NS + s
    base0 = wid * CH_T * CHUNK

    pltpu.sync_copy(ones_hbm, ones_v)

    @pl.when(s == 0)
    def _():
        pltpu.sync_copy(zeros_hbm, deg_sh)
    plsc.subcore_barrier()

    bufs = ((idx0, isem0, ssem0), (idx1, isem1, ssem1))

    @pl.loop(0, (CH_T + 1) // 2)
    def _(i):
        for b in (0, 1):
            idx_v, isem, ssem = bufs[b]
            j = 2 * i + b

            @pl.when(j < CH_T)
            def _():
                base = base0 + j * CHUNK

                @pl.when(j >= 2)
                def _():
                    pltpu.make_async_copy(ones_v, deg_sh.at[idx_v], ssem).wait()

                pltpu.async_copy(dst_hbm.at[pl.ds(base, CHUNK)],
                                 idx_v, isem).wait()
                pltpu.async_copy(ones_v, deg_sh.at[idx_v], ssem, add=True)

    pltpu.make_async_copy(ones_v, deg_sh.at[idx0], ssem0).wait()
    pltpu.make_async_copy(ones_v, deg_sh.at[idx1], ssem1).wait()

    plsc.subcore_barrier()

    @pl.when(s == 0)
    def _():
        pltpu.sync_copy(deg_sh, out_hbm.at[c])


@functools.partial(
    pl.kernel,
    out_type=jax.ShapeDtypeStruct((NC, N, D), jnp.float32),
    mesh=_mesh,
    scratch_types=[
        pltpu.VMEM((MCH,), jnp.int32),
        pltpu.VMEM((MCH,), jnp.int32),
        pltpu.VMEM((MCH,), jnp.int32),
        pltpu.VMEM((MCH,), jnp.int32),
        pltpu.VMEM((MCH,), jnp.int32),
        pltpu.VMEM((MCH,), jnp.int32),
        pltpu.VMEM((MCH, D), jnp.float32),
        pltpu.VMEM((MCH, D), jnp.float32),
        pltpu.VMEM_SHARED((N, D), jnp.float32),
        pltpu.SemaphoreType.DMA,
        pltpu.SemaphoreType.DMA,
        pltpu.SemaphoreType.DMA,
        pltpu.SemaphoreType.DMA,
        pltpu.SemaphoreType.DMA,
        pltpu.SemaphoreType.DMA,
    ],
)
def _sc_scatter(hs_hbm, src_hbm, dst_hbm, zeros_hbm, out_hbm,
                src0, src1, dstA, dstB, dstC, dstD, rows0, rows1, acc_sh,
                isem0, gsem0, ssem0, isem1, gsem1, ssem1):
    c = lax.axis_index("c")
    s = lax.axis_index("s")
    wid = c * NS + s

    @pl.when(s == 0)
    def _():
        pltpu.sync_copy(zeros_hbm, acc_sh)
    plsc.subcore_barrier()

    srcs = (src0, src1)
    dsts = (dstA, dstB, dstC, dstD)
    rows = (rows0, rows1)
    isems = (isem0, isem1)
    gsems = (gsem0, gsem1)
    ssems = (ssem0, ssem1)

    def _base(k):
        return (wid + (NC * NS) * k) * MCH

    for k0 in (0, 1):
        pltpu.async_copy(src_hbm.at[pl.ds(_base(k0), MCH)],
                         srcs[k0], isems[k0])
        pltpu.async_copy(dst_hbm.at[pl.ds(_base(k0), MCH)],
                         dsts[k0], isems[k0])

    @pl.loop(0, (KT + 1) // 2)
    def _(i):
        for b in (0, 1):
            src_v, rows_v = srcs[b], rows[b]
            isem, gsem, ssem = isems[b], gsems[b], ssems[b]
            k = 2 * i + b
            ch = wid + (NC * NS) * k

            @pl.when(ch < NCHM)
            def _():
                base = _base(k)
                pltpu.make_async_copy(src_hbm.at[pl.ds(base, MCH)],
                                      src_v, isem).wait()
                for q in (0, 1, 2, 3):
                    @pl.when((k % 4) == q)
                    def _(q=q):
                        dq = dsts[q]
                        pltpu.make_async_copy(dst_hbm.at[pl.ds(base, MCH)],
                                              dq, isem).wait()

                        @pl.when(k >= 2)
                        def _():
                            pltpu.make_async_copy(
                                rows_v, acc_sh.at[dq], ssem).wait()

                        ch2 = wid + (NC * NS) * (k + 2)

                        @pl.when(ch2 < NCHM)
                        def _():
                            pltpu.async_copy(
                                dst_hbm.at[pl.ds(_base(k + 2), MCH)],
                                dsts[(q + 2) % 4], isem)

                        pltpu.async_copy(hs_hbm.at[src_v], rows_v, gsem).wait()

                        @pl.when(ch2 < NCHM)
                        def _():
                            pltpu.async_copy(
                                src_hbm.at[pl.ds(_base(k + 2), MCH)],
                                src_v, isem)

                        pltpu.async_copy(rows_v, acc_sh.at[dq], ssem, add=True)

    pltpu.make_async_copy(rows0, acc_sh.at[dstA], ssem0).wait()
    pltpu.make_async_copy(rows1, acc_sh.at[dstB], ssem1).wait()

    plsc.subcore_barrier()

    @pl.when(s == 0)
    def _():
        pltpu.sync_copy(acc_sh, out_hbm.at[c])


def _dot(a, b):
    return lax.dot_general(a, b, (((1,), (0,)), ((), ())),
                           precision=lax.Precision.HIGHEST,
                           preferred_element_type=jnp.float32)


def _mm_body(x_ref, w_ref, o_ref):
    o_ref[...] = _dot(x_ref[...], w_ref[...])


_mm = pl.pallas_call(
    _mm_body,
    grid=(NB,),
    in_specs=[pl.BlockSpec((RB, D), lambda i: (i, 0)),
              pl.BlockSpec((D, D), lambda i: (0, 0))],
    out_specs=pl.BlockSpec((RB, D), lambda i: (i, 0)),
    out_shape=jax.ShapeDtypeStruct((N, D), jnp.float32),
)


def _scale_body(h_ref, deg_ref, o_ref):
    dinv = lax.rsqrt(jnp.maximum(deg_ref[...] + 1.0, 1.0))
    o_ref[...] = h_ref[...] * dinv


_scale = pl.pallas_call(
    _scale_body,
    grid=(NB,),
    in_specs=[pl.BlockSpec((RB, D), lambda i: (i, 0)),
              pl.BlockSpec((RB, 1), lambda i: (i, 0))],
    out_specs=pl.BlockSpec((RB, D), lambda i: (i, 0)),
    out_shape=jax.ShapeDtypeStruct((N, D), jnp.float32),
)


def _final_body(x_ref, hs_ref, accp_ref, deg_ref, bg_ref, w1_ref, b1_ref,
                w2_ref, b2_ref, g1_ref, bb1_ref, g2_ref, bb2_ref, o_ref):
    dinv = lax.rsqrt(jnp.maximum(deg_ref[...] + 1.0, 1.0))
    g = (accp_ref[0] + accp_ref[1] + hs_ref[...]) * dinv + bg_ref[...]
    x1 = x_ref[...] + g
    mu = jnp.mean(x1, axis=1, keepdims=True)
    var = jnp.mean(jnp.square(x1 - mu), axis=1, keepdims=True)
    xn = (x1 - mu) * lax.rsqrt(var + 1e-5) * g1_ref[...] + bb1_ref[...]
    t = jnp.maximum(_dot(xn, w1_ref[...]) + b1_ref[...], 0.0)
    x2 = xn + _dot(t, w2_ref[...]) + b2_ref[...]
    mu2 = jnp.mean(x2, axis=1, keepdims=True)
    var2 = jnp.mean(jnp.square(x2 - mu2), axis=1, keepdims=True)
    o_ref[...] = (x2 - mu2) * lax.rsqrt(var2 + 1e-5) * g2_ref[...] + bb2_ref[...]


_final = pl.pallas_call(
    _final_body,
    grid=(NB,),
    in_specs=[pl.BlockSpec((RB, D), lambda i: (i, 0)),
              pl.BlockSpec((RB, D), lambda i: (i, 0)),
              pl.BlockSpec((NC, RB, D), lambda i: (0, i, 0)),
              pl.BlockSpec((RB, 1), lambda i: (i, 0)),
              pl.BlockSpec((1, D), lambda i: (0, 0)),
              pl.BlockSpec((D, FF), lambda i: (0, 0)),
              pl.BlockSpec((1, FF), lambda i: (0, 0)),
              pl.BlockSpec((FF, D), lambda i: (0, 0)),
              pl.BlockSpec((1, D), lambda i: (0, 0)),
              pl.BlockSpec((1, D), lambda i: (0, 0)),
              pl.BlockSpec((1, D), lambda i: (0, 0)),
              pl.BlockSpec((1, D), lambda i: (0, 0)),
              pl.BlockSpec((1, D), lambda i: (0, 0))],
    out_specs=pl.BlockSpec((RB, D), lambda i: (i, 0)),
    out_shape=jax.ShapeDtypeStruct((N, D), jnp.float32),
)


def kernel(x, edge_index, W_gnn, b_gnn, W1, b1, W2, b2,
           ln1_g, ln1_b, ln2_g, ln2_b):
    src = edge_index[0]
    dst = edge_index[1]
    zeros_nd = jnp.zeros((N, D), jnp.float32)
    degp = _sc_degree(dst, jnp.ones((CHUNK,), jnp.float32),
                      jnp.zeros((N,), jnp.float32))
    deg2d = (degp[0] + degp[1]).reshape(N, 1)
    h = _mm(x, W_gnn)
    hs = _scale(h, deg2d)
    accp = _sc_scatter(hs, src, dst, zeros_nd)
    return _final(x, hs, accp, deg2d, b_gnn.reshape(1, D), W1,
                  b1.reshape(1, FF), W2, b2.reshape(1, D),
                  ln1_g.reshape(1, D), ln1_b.reshape(1, D),
                  ln2_g.reshape(1, D), ln2_b.reshape(1, D))
